# Initial kernel scaffold; baseline (speedup 1.0000x reference)
import jax
import jax.numpy as jnp
from jax.experimental import pallas as pl


def kernel(x, edge_index, lap_pe, edge_weight, W1, b1, W2, b2, Wmu, bmu, Wlv, blv):
    raise NotImplementedError("write your pallas kernel here")


# speedup vs baseline: 7.8290x; 7.8290x over previous
import functools

import jax
import jax.numpy as jnp
from jax import lax
from jax.experimental import pallas as pl
from jax.experimental.pallas import tpu as pltpu
from jax.experimental.pallas import tpu_sc as plsc

N_NODES = 10000
N_PAD = 10240
NUM_CORES = 2
NUM_SUBCORES = 16
NW = NUM_CORES * NUM_SUBCORES
BATCH = 128
ROWS_PER_SUB = N_PAD // NUM_SUBCORES


def _z():
    return jnp.int32(0)


def _im32(f):
    import functools as _ft

    @_ft.wraps(f)
    def g(*args):
        return tuple(jnp.asarray(v, jnp.int32) for v in f(*args))

    return g


def _mesh():
    return plsc.VectorSubcoreMesh(core_axis_name="c", subcore_axis_name="s")


def _wid():
    return lax.axis_index("s") * jnp.int32(NUM_CORES) + lax.axis_index("c")


def _deg_partials(src3, w3, nb):
    @functools.partial(
        pl.kernel,
        mesh=_mesh(),
        compiler_params=pltpu.CompilerParams(needs_layout_passes=False),
        out_type=jax.ShapeDtypeStruct((NUM_CORES, N_PAD), jnp.float32),
        scratch_types=[
            pltpu.VMEM((nb, BATCH), jnp.int32),
            pltpu.VMEM((nb, BATCH), jnp.float32),
            pltpu.VMEM((ROWS_PER_SUB,), jnp.float32),
            pltpu.VMEM_SHARED((N_PAD,), jnp.float32),
        ],
    )
    def k(src_h, w_h, out_h, src_v, w_v, z_v, deg_sp):
        sid = lax.axis_index("s")
        cid = lax.axis_index("c")
        wid = _wid()
        pltpu.sync_copy(src_h.at[wid], src_v)
        pltpu.sync_copy(w_h.at[wid], w_v)

        zero16 = jnp.zeros((16,), jnp.float32)

        def zfill(i, carry):
            z_v[pl.ds(i * jnp.int32(16), 16)] = zero16
            return carry

        lax.fori_loop(jnp.int32(0), jnp.int32(ROWS_PER_SUB // 16), zfill, jnp.int32(0))
        pltpu.sync_copy(z_v, deg_sp.at[pl.ds(sid * jnp.int32(ROWS_PER_SUB), ROWS_PER_SUB)])
        plsc.subcore_barrier()

        def jbody(j, carry):
            pltpu.sync_copy(w_v.at[j], deg_sp.at[src_v.at[j]], add=True)
            return carry

        lax.fori_loop(jnp.int32(0), jnp.int32(nb), jbody, jnp.int32(0))
        plsc.subcore_barrier()
        pltpu.sync_copy(
            deg_sp.at[pl.ds(sid * jnp.int32(ROWS_PER_SUB), ROWS_PER_SUB)],
            out_h.at[cid, pl.ds(sid * jnp.int32(ROWS_PER_SUB), ROWS_PER_SUB)],
        )

    return k(src3, w3)


def _dis_tc(degp):
    def body(d_ref, o_ref):
        deg = d_ref[0, :] + d_ref[1, :]
        o_ref[0, :] = jnp.where(deg > 0, lax.rsqrt(deg), 0.0)

    return pl.pallas_call(
        body, out_shape=jax.ShapeDtypeStruct((1, N_PAD), jnp.float32)
    )(degp)


def _what_sc(src3, dst3, w3, dis, nb):
    @functools.partial(
        pl.kernel,
        mesh=_mesh(),
        compiler_params=pltpu.CompilerParams(needs_layout_passes=False),
        out_type=jax.ShapeDtypeStruct((NW, nb, BATCH), jnp.float32),
        scratch_types=[
            pltpu.VMEM((N_PAD,), jnp.float32),
            pltpu.VMEM((nb, BATCH), jnp.int32),
            pltpu.VMEM((nb, BATCH), jnp.int32),
            pltpu.VMEM((nb, BATCH), jnp.float32),
            pltpu.VMEM((nb, BATCH), jnp.float32),
        ],
    )
    def k(src_h, dst_h, w_h, dis_h, out_h, dis_v, src_v, dst_v, w_v, o_v):
        wid = _wid()
        pltpu.sync_copy(dis_h, dis_v)
        pltpu.sync_copy(src_h.at[wid], src_v)
        pltpu.sync_copy(dst_h.at[wid], dst_v)
        pltpu.sync_copy(w_h.at[wid], w_v)

        def jbody(j, carry):
            for q in range(BATCH // 16):
                sl = pl.ds(q * 16, 16)
                s16 = src_v[j, sl]
                d16 = dst_v[j, sl]
                wv = w_v[j, sl]
                sv = plsc.load_gather(dis_v, [s16])
                dv = plsc.load_gather(dis_v, [d16])
                o_v[j, sl] = -(sv * wv * dv)
            return carry

        lax.fori_loop(jnp.int32(0), jnp.int32(nb), jbody, jnp.int32(0))
        pltpu.sync_copy(o_v, out_h.at[wid])

    return k(src3, dst3, w3, dis)


def _matvec_sc(t, src3, dst3, what3, nb, d):
    ch_n = 27
    assert nb % ch_n == 0
    nch = nb // ch_n

    @functools.partial(
        pl.kernel,
        mesh=_mesh(),
        compiler_params=pltpu.CompilerParams(
            needs_layout_passes=False, use_tc_tiling_on_sc=False
        ),
        out_type=jax.ShapeDtypeStruct((NUM_CORES, N_PAD, d), jnp.float32),
        scratch_types=[
            pltpu.VMEM((ch_n, BATCH), jnp.int32),
            pltpu.VMEM((ch_n, BATCH), jnp.int32),
            pltpu.VMEM((ch_n, BATCH), jnp.float32),
            pltpu.VMEM((BATCH, d), jnp.float32),
            pltpu.VMEM_SHARED((N_PAD, d), jnp.float32),
            pltpu.SemaphoreType.DMA,
        ],
    )
    def k(t_h, src_h, dst_h, w_h, out_h, src_v, dst_v, w_v, rows_v, acc_sp, sem):
        sid = lax.axis_index("s")
        cid = lax.axis_index("c")
        wid = _wid()

        zero16 = jnp.zeros((16,), jnp.float32)

        def zfill(r, carry):
            for q in range(d // 16):
                rows_v[r, pl.ds(q * 16, 16)] = zero16
            return carry

        lax.fori_loop(jnp.int32(0), jnp.int32(BATCH), zfill, jnp.int32(0))

        def zcopy(m, carry):
            pltpu.sync_copy(
                rows_v,
                acc_sp.at[
                    pl.ds(sid * jnp.int32(ROWS_PER_SUB) + m * jnp.int32(BATCH), BATCH)
                ],
            )
            return carry

        lax.fori_loop(
            jnp.int32(0), jnp.int32(ROWS_PER_SUB // BATCH), zcopy, jnp.int32(0)
        )
        plsc.subcore_barrier()

        def cbody(c, carry):
            csl = pl.ds(c * jnp.int32(ch_n), ch_n)
            pltpu.sync_copy(src_h.at[wid, csl], src_v)
            pltpu.sync_copy(dst_h.at[wid, csl], dst_v)
            pltpu.sync_copy(w_h.at[wid, csl], w_v)

            def jbody(j, carry2):
                pltpu.async_copy(t_h.at[src_v.at[j]], rows_v, sem).wait()

                def ebody(e, c2):
                    wb = plsc.load_gather(
                        w_v,
                        [jnp.full((16,), j, jnp.int32), jnp.full((16,), e, jnp.int32)],
                    )
                    for q in range(d // 16):
                        sl = pl.ds(q * 16, 16)
                        rows_v[e, sl] = rows_v[e, sl] * wb
                    return c2

                lax.fori_loop(jnp.int32(0), jnp.int32(BATCH), ebody, jnp.int32(0))
                pltpu.sync_copy(rows_v, acc_sp.at[dst_v.at[j]], add=True)
                return carry2

            lax.fori_loop(jnp.int32(0), jnp.int32(ch_n), jbody, jnp.int32(0))
            return carry

        lax.fori_loop(jnp.int32(0), jnp.int32(nch), cbody, jnp.int32(0))
        plsc.subcore_barrier()
        pltpu.sync_copy(
            acc_sp.at[pl.ds(sid * jnp.int32(ROWS_PER_SUB), ROWS_PER_SUB)],
            out_h.at[cid, pl.ds(sid * jnp.int32(ROWS_PER_SUB), ROWS_PER_SUB)],
        )

    return k(t, src3, dst3, what3)


def _combine0_tc(parts, d):
    blk = 1024

    def body(p_ref, o_ref):
        o_ref[...] = p_ref[0] + p_ref[1]

    return pl.pallas_call(
        body,
        out_shape=jax.ShapeDtypeStruct((N_PAD, d), jnp.float32),
        grid=(N_PAD // blk,),
        in_specs=[pl.BlockSpec((2, blk, d), _im32(lambda i: (0, i, 0)))],
        out_specs=pl.BlockSpec((blk, d), _im32(lambda i: (i, 0))),
    )(parts)


def _combine_tc(parts, tprev, d):
    blk = 1024

    def body(p_ref, tp_ref, o_ref):
        o_ref[...] = 2.0 * (p_ref[0] + p_ref[1]) - tp_ref[...]

    return pl.pallas_call(
        body,
        out_shape=jax.ShapeDtypeStruct((N_PAD, d), jnp.float32),
        grid=(N_PAD // blk,),
        in_specs=[
            pl.BlockSpec((2, blk, d), _im32(lambda i: (0, i, 0))),
            pl.BlockSpec((blk, d), _im32(lambda i: (i, 0))),
        ],
        out_specs=pl.BlockSpec((blk, d), _im32(lambda i: (i, 0))),
    )(parts, tprev)


def _layer1_tc(t0, t1, t2, t3, W, b2d, din):
    blk = 512

    def body(t0r, t1r, t2r, t3r, w_r, b_r, o_ref):
        acc = jnp.dot(t0r[...], w_r[0], preferred_element_type=jnp.float32)
        acc = acc + jnp.dot(t1r[...], w_r[1], preferred_element_type=jnp.float32)
        acc = acc + jnp.dot(t2r[...], w_r[2], preferred_element_type=jnp.float32)
        acc = acc + jnp.dot(t3r[...], w_r[3], preferred_element_type=jnp.float32)
        o_ref[...] = jnp.maximum(acc + b_r[...], 0.0)

    tspec = pl.BlockSpec((blk, din), _im32(lambda i: (i, 0)))
    return pl.pallas_call(
        body,
        out_shape=jax.ShapeDtypeStruct((N_PAD, 128), jnp.float32),
        grid=(N_PAD // blk,),
        in_specs=[
            tspec,
            tspec,
            tspec,
            tspec,
            pl.BlockSpec((4, din, 128), _im32(lambda i: (0, 0, 0))),
            pl.BlockSpec((1, 128), _im32(lambda i: (0, 0))),
        ],
        out_specs=pl.BlockSpec((blk, 128), _im32(lambda i: (i, 0))),
    )(t0, t1, t2, t3, W, b2d)


def _layer2_tc(t0, t1, t2, t3, W, b2d, Wmu, bmu2d, Wlv, blv2d):
    blk = 512
    nblk = N_PAD // blk
    lat = Wmu.shape[1]

    def body(t0r, t1r, t2r, t3r, w_r, b_r, wmu_r, bmu_r, wlv_r, blv_r,
             mu_ref, lv_ref, acc_ref):
        i = pl.program_id(0)

        @pl.when(i == 0)
        def _():
            acc_ref[...] = jnp.zeros_like(acc_ref)

        acc = jnp.dot(t0r[...], w_r[0], preferred_element_type=jnp.float32)
        acc = acc + jnp.dot(t1r[...], w_r[1], preferred_element_type=jnp.float32)
        acc = acc + jnp.dot(t2r[...], w_r[2], preferred_element_type=jnp.float32)
        acc = acc + jnp.dot(t3r[...], w_r[3], preferred_element_type=jnp.float32)
        h = jnp.maximum(acc + b_r[...], 0.0)
        row = i * blk + lax.broadcasted_iota(jnp.int32, (blk, 1), 0)
        h = jnp.where(row < N_NODES, h, 0.0)
        acc_ref[...] = acc_ref[...] + jnp.sum(h, axis=0, keepdims=True)

        @pl.when(i == nblk - 1)
        def _():
            ge = acc_ref[...] * (1.0 / N_NODES)
            mu_ref[...] = (
                jnp.dot(ge, wmu_r[...], preferred_element_type=jnp.float32)
                + bmu_r[...]
            )
            lv_ref[...] = (
                jnp.dot(ge, wlv_r[...], preferred_element_type=jnp.float32)
                + blv_r[...]
            )

    tspec = pl.BlockSpec((blk, 128), _im32(lambda i: (i, 0)))
    return pl.pallas_call(
        body,
        out_shape=(
            jax.ShapeDtypeStruct((1, lat), jnp.float32),
            jax.ShapeDtypeStruct((1, lat), jnp.float32),
        ),
        grid=(nblk,),
        in_specs=[
            tspec,
            tspec,
            tspec,
            tspec,
            pl.BlockSpec((4, 128, 128), _im32(lambda i: (0, 0, 0))),
            pl.BlockSpec((1, 128), _im32(lambda i: (0, 0))),
            pl.BlockSpec((128, lat), _im32(lambda i: (0, 0))),
            pl.BlockSpec((1, lat), _im32(lambda i: (0, 0))),
            pl.BlockSpec((128, lat), _im32(lambda i: (0, 0))),
            pl.BlockSpec((1, lat), _im32(lambda i: (0, 0))),
        ],
        out_specs=(
            pl.BlockSpec((1, lat), _im32(lambda i: (0, 0))),
            pl.BlockSpec((1, lat), _im32(lambda i: (0, 0))),
        ),
        scratch_shapes=[pltpu.VMEM((1, 128), jnp.float32)],
    )(t0, t1, t2, t3, W, b2d, Wmu, bmu2d, Wlv, blv2d)


def kernel(x, edge_index, lap_pe, edge_weight, W1, b1, W2, b2, Wmu, bmu, Wlv, blv):
    n = x.shape[0]
    e = edge_weight.shape[0]

    src = edge_index[0].astype(jnp.int32)
    dst = edge_index[1].astype(jnp.int32)
    loop = jnp.arange(n, dtype=jnp.int32)
    src = jnp.concatenate([src, loop])
    dst = jnp.concatenate([dst, loop])
    w = jnp.concatenate([edge_weight.astype(jnp.float32), jnp.ones((n,), jnp.float32)])

    ep = e + n
    nb = -(-ep // (NW * BATCH))
    pad = NW * nb * BATCH - ep
    src = jnp.concatenate([src, jnp.zeros((pad,), jnp.int32)])
    dst = jnp.concatenate([dst, jnp.full((pad,), N_NODES, jnp.int32)])
    w = jnp.concatenate([w, jnp.zeros((pad,), jnp.float32)])
    src3 = src.reshape(NW, nb, BATCH)
    dst3 = dst.reshape(NW, nb, BATCH)
    w3 = w.reshape(NW, nb, BATCH)

    degp = _deg_partials(src3, w3, nb)
    dis = _dis_tc(degp).reshape(N_PAD)
    what3 = _what_sc(src3, dst3, w3, dis, nb)

    xc = jnp.concatenate([x.astype(jnp.float32), lap_pe.astype(jnp.float32)], axis=1)
    din = xc.shape[1]
    t0 = jnp.pad(xc, ((0, N_PAD - n), (0, 0)))

    t1 = _combine0_tc(_matvec_sc(t0, src3, dst3, what3, nb, din), din)
    t2 = _combine_tc(_matvec_sc(t1, src3, dst3, what3, nb, din), t0, din)
    t3 = _combine_tc(_matvec_sc(t2, src3, dst3, what3, nb, din), t1, din)
    h = _layer1_tc(t0, t1, t2, t3, W1, b1.reshape(1, -1), din)

    u1 = _combine0_tc(_matvec_sc(h, src3, dst3, what3, nb, 128), 128)
    u2 = _combine_tc(_matvec_sc(u1, src3, dst3, what3, nb, 128), h, 128)
    u3 = _combine_tc(_matvec_sc(u2, src3, dst3, what3, nb, 128), u1, 128)

    mu, logvar = _layer2_tc(
        h, u1, u2, u3, W2, b2.reshape(1, -1),
        Wmu, bmu.reshape(1, -1), Wlv, blv.reshape(1, -1),
    )
    return (mu, logvar)


# speedup vs baseline: 10.5650x; 1.3495x over previous
import functools

import jax
import jax.numpy as jnp
import numpy as np
from jax import lax
from jax.experimental import pallas as pl
from jax.experimental.pallas import tpu as pltpu
from jax.experimental.pallas import tpu_sc as plsc

N_NODES = 10000
N_PAD = 10240
NUM_CORES = 2
NUM_SUBCORES = 16
NW = NUM_CORES * NUM_SUBCORES
BATCH = 96
ROWS_PER_SUB = N_PAD // NUM_SUBCORES


def _z():
    return jnp.int32(0)


def _im32(f):
    import functools as _ft

    @_ft.wraps(f)
    def g(*args):
        return tuple(jnp.asarray(v, jnp.int32) for v in f(*args))

    return g


def _mesh():
    return plsc.VectorSubcoreMesh(core_axis_name="c", subcore_axis_name="s")


def _wid():
    return lax.axis_index("s") * jnp.int32(NUM_CORES) + lax.axis_index("c")


def _deg_partials(src3, w3, nb):
    @functools.partial(
        pl.kernel,
        mesh=_mesh(),
        compiler_params=pltpu.CompilerParams(needs_layout_passes=False),
        out_type=jax.ShapeDtypeStruct((NUM_CORES, N_PAD), jnp.float32),
        scratch_types=[
            pltpu.VMEM((nb, BATCH), jnp.int32),
            pltpu.VMEM((nb, BATCH), jnp.float32),
            pltpu.VMEM((ROWS_PER_SUB,), jnp.float32),
            pltpu.VMEM_SHARED((N_PAD,), jnp.float32),
        ],
    )
    def k(src_h, w_h, out_h, src_v, w_v, z_v, deg_sp):
        sid = lax.axis_index("s")
        cid = lax.axis_index("c")
        wid = _wid()
        pltpu.sync_copy(src_h.at[wid], src_v)
        pltpu.sync_copy(w_h.at[wid], w_v)

        zero16 = jnp.zeros((16,), jnp.float32)

        def zfill(i, carry):
            z_v[pl.ds(i * jnp.int32(16), 16)] = zero16
            return carry

        lax.fori_loop(jnp.int32(0), jnp.int32(ROWS_PER_SUB // 16), zfill, jnp.int32(0))
        pltpu.sync_copy(z_v, deg_sp.at[pl.ds(sid * jnp.int32(ROWS_PER_SUB), ROWS_PER_SUB)])
        plsc.subcore_barrier()

        def jbody(j, carry):
            pltpu.sync_copy(w_v.at[j], deg_sp.at[src_v.at[j]], add=True)
            return carry

        lax.fori_loop(jnp.int32(0), jnp.int32(nb), jbody, jnp.int32(0))
        plsc.subcore_barrier()
        pltpu.sync_copy(
            deg_sp.at[pl.ds(sid * jnp.int32(ROWS_PER_SUB), ROWS_PER_SUB)],
            out_h.at[cid, pl.ds(sid * jnp.int32(ROWS_PER_SUB), ROWS_PER_SUB)],
        )

    return k(src3, w3)


def _dis_tc(degp):
    def body(d_ref, o_ref):
        deg = d_ref[0, :] + d_ref[1, :]
        o_ref[0, :] = jnp.where(deg > 0, lax.rsqrt(deg), 0.0)

    return pl.pallas_call(
        body, out_shape=jax.ShapeDtypeStruct((1, N_PAD), jnp.float32)
    )(degp)


def _what_sc(src3, dst3, w3, dis, nb):
    @functools.partial(
        pl.kernel,
        mesh=_mesh(),
        compiler_params=pltpu.CompilerParams(needs_layout_passes=False),
        out_type=jax.ShapeDtypeStruct((NW, nb, BATCH), jnp.float32),
        scratch_types=[
            pltpu.VMEM((N_PAD,), jnp.float32),
            pltpu.VMEM((nb, BATCH), jnp.int32),
            pltpu.VMEM((nb, BATCH), jnp.int32),
            pltpu.VMEM((nb, BATCH), jnp.float32),
            pltpu.VMEM((nb, BATCH), jnp.float32),
        ],
    )
    def k(src_h, dst_h, w_h, dis_h, out_h, dis_v, src_v, dst_v, w_v, o_v):
        wid = _wid()
        pltpu.sync_copy(dis_h, dis_v)
        pltpu.sync_copy(src_h.at[wid], src_v)
        pltpu.sync_copy(dst_h.at[wid], dst_v)
        pltpu.sync_copy(w_h.at[wid], w_v)

        def jbody(j, carry):
            for q in range(BATCH // 16):
                sl = pl.ds(q * 16, 16)
                s16 = src_v[j, sl]
                d16 = dst_v[j, sl]
                wv = w_v[j, sl]
                sv = plsc.load_gather(dis_v, [s16])
                dv = plsc.load_gather(dis_v, [d16])
                o_v[j, sl] = -(sv * wv * dv)
            return carry

        lax.fori_loop(jnp.int32(0), jnp.int32(nb), jbody, jnp.int32(0))
        pltpu.sync_copy(o_v, out_h.at[wid])

    return k(src3, dst3, w3, dis)


def _matvec_sc(t, src3, dst3, what3, nb, d):
    ch_n = 36
    assert nb % ch_n == 0 and ch_n % 2 == 0
    npair = nb // 2
    pch = ch_n // 2
    ng = BATCH // 16
    i32 = jnp.int32

    @functools.partial(
        pl.kernel,
        mesh=_mesh(),
        compiler_params=pltpu.CompilerParams(
            needs_layout_passes=False, use_tc_tiling_on_sc=False
        ),
        out_type=jax.ShapeDtypeStruct((NUM_CORES, N_PAD, d), jnp.float32),
        scratch_types=[
            pltpu.VMEM((ch_n, BATCH), jnp.int32),
            pltpu.VMEM((ch_n, BATCH), jnp.int32),
            pltpu.VMEM((ch_n, BATCH), jnp.float32),
            pltpu.VMEM((BATCH, d), jnp.float32),
            pltpu.VMEM((BATCH, d), jnp.float32),
            pltpu.VMEM_SHARED((N_PAD, d), jnp.float32),
            pltpu.SemaphoreType.DMA,
            pltpu.SemaphoreType.DMA,
            pltpu.SemaphoreType.DMA,
            pltpu.SemaphoreType.DMA,
        ],
    )
    def k(t_h, src_h, dst_h, w_h, out_h,
          src_v, dst_v, w_v, rows_a, rows_b, acc_sp, gsa, gsb, ssa, ssb):
        sid = lax.axis_index("s")
        cid = lax.axis_index("c")
        wid = _wid()

        zero16 = jnp.zeros((16,), jnp.float32)

        def zfill(r, carry):
            for q in range(d // 16):
                rows_a[r, pl.ds(q * 16, 16)] = zero16
            return carry

        lax.fori_loop(i32(0), i32(BATCH), zfill, i32(0))

        zr = 80
        assert ROWS_PER_SUB % zr == 0 and zr <= BATCH

        def zcopy(m, carry):
            pltpu.sync_copy(
                rows_a.at[pl.ds(i32(0), zr)],
                acc_sp.at[pl.ds(sid * i32(ROWS_PER_SUB) + m * i32(zr), zr)],
            )
            return carry

        lax.fori_loop(i32(0), i32(ROWS_PER_SUB // zr), zcopy, i32(0))
        plsc.subcore_barrier()

        def load_chunk(c):
            csl = pl.ds(c * i32(ch_n), ch_n)
            pltpu.sync_copy(src_h.at[wid, csl], src_v)
            pltpu.sync_copy(dst_h.at[wid, csl], dst_v)
            pltpu.sync_copy(w_h.at[wid, csl], w_v)

        def g_start(jj, rows, sem):
            pltpu.async_copy(t_h.at[src_v.at[jj]], rows, sem)

        def g_wait(jj, rows, sem):
            pltpu.make_async_copy(t_h.at[src_v.at[jj]], rows, sem).wait()

        def s_start(jj, rows, sem):
            pltpu.async_copy(rows, acc_sp.at[dst_v.at[jj]], sem, add=True)

        def s_wait(jj, rows, sem):
            pltpu.make_async_copy(rows, acc_sp.at[dst_v.at[jj]], sem).wait()

        zi16 = lax.broadcasted_iota(jnp.int32, (16,), 0) * i32(0)

        def scale(rows, jj):
            idx_j = zi16 + jj

            def gbody(g, carry):
                g16 = g * i32(16)
                for e in range(16):
                    r = g16 + i32(e)
                    wb = plsc.load_gather(w_v, [idx_j, zi16 + r])
                    for q in range(d // 16):
                        sl = pl.ds(q * 16, 16)
                        rows[r, sl] = rows[r, sl] * wb
                return carry

            lax.fori_loop(i32(0), i32(ng), gbody, i32(0))

        load_chunk(i32(0))
        g_start(i32(0), rows_a, gsa)

        def pbody(p, carry):
            m = lax.rem(p, i32(pch))
            je = m * i32(2)

            @pl.when(jnp.logical_and(m == 0, p > 0))
            def _():
                s_wait(i32(ch_n - 1), rows_b, ssb)
                s_wait(i32(ch_n - 2), rows_a, ssa)
                load_chunk(lax.div(p, i32(pch)))
                g_start(i32(0), rows_a, gsa)

            @pl.when(jnp.logical_and(m != 0, p > 0))
            def _():
                s_wait(je - i32(1), rows_b, ssb)

            g_start(je + i32(1), rows_b, gsb)
            g_wait(je, rows_a, gsa)
            scale(rows_a, je)
            s_start(je, rows_a, ssa)
            g_wait(je + i32(1), rows_b, gsb)

            @pl.when(m != i32(pch - 1))
            def _():
                s_wait(je, rows_a, ssa)
                g_start(je + i32(2), rows_a, gsa)

            scale(rows_b, je + i32(1))
            s_start(je + i32(1), rows_b, ssb)
            return carry

        lax.fori_loop(i32(0), i32(npair), pbody, i32(0))
        s_wait(i32(ch_n - 2), rows_a, ssa)
        s_wait(i32(ch_n - 1), rows_b, ssb)
        plsc.subcore_barrier()
        pltpu.sync_copy(
            acc_sp.at[pl.ds(sid * i32(ROWS_PER_SUB), ROWS_PER_SUB)],
            out_h.at[cid, pl.ds(sid * i32(ROWS_PER_SUB), ROWS_PER_SUB)],
        )

    return k(t, src3, dst3, what3)


def _combine0_tc(parts, d):
    blk = 1024

    def body(p_ref, o_ref):
        o_ref[...] = p_ref[0] + p_ref[1]

    return pl.pallas_call(
        body,
        out_shape=jax.ShapeDtypeStruct((N_PAD, d), jnp.float32),
        grid=(N_PAD // blk,),
        in_specs=[pl.BlockSpec((2, blk, d), _im32(lambda i: (0, i, 0)))],
        out_specs=pl.BlockSpec((blk, d), _im32(lambda i: (i, 0))),
    )(parts)


def _combine_tc(parts, tprev, d):
    blk = 1024

    def body(p_ref, tp_ref, o_ref):
        o_ref[...] = 2.0 * (p_ref[0] + p_ref[1]) - tp_ref[...]

    return pl.pallas_call(
        body,
        out_shape=jax.ShapeDtypeStruct((N_PAD, d), jnp.float32),
        grid=(N_PAD // blk,),
        in_specs=[
            pl.BlockSpec((2, blk, d), _im32(lambda i: (0, i, 0))),
            pl.BlockSpec((blk, d), _im32(lambda i: (i, 0))),
        ],
        out_specs=pl.BlockSpec((blk, d), _im32(lambda i: (i, 0))),
    )(parts, tprev)


def _layer1_tc(t0, t1, t2, t3, W, b2d, din):
    blk = 512

    def body(t0r, t1r, t2r, t3r, w_r, b_r, o_ref):
        acc = jnp.dot(t0r[...], w_r[0], preferred_element_type=jnp.float32)
        acc = acc + jnp.dot(t1r[...], w_r[1], preferred_element_type=jnp.float32)
        acc = acc + jnp.dot(t2r[...], w_r[2], preferred_element_type=jnp.float32)
        acc = acc + jnp.dot(t3r[...], w_r[3], preferred_element_type=jnp.float32)
        o_ref[...] = jnp.maximum(acc + b_r[...], 0.0)

    tspec = pl.BlockSpec((blk, din), _im32(lambda i: (i, 0)))
    return pl.pallas_call(
        body,
        out_shape=jax.ShapeDtypeStruct((N_PAD, 128), jnp.float32),
        grid=(N_PAD // blk,),
        in_specs=[
            tspec,
            tspec,
            tspec,
            tspec,
            pl.BlockSpec((4, din, 128), _im32(lambda i: (0, 0, 0))),
            pl.BlockSpec((1, 128), _im32(lambda i: (0, 0))),
        ],
        out_specs=pl.BlockSpec((blk, 128), _im32(lambda i: (i, 0))),
    )(t0, t1, t2, t3, W, b2d)


def _layer2_tc(t0, t1, t2, t3, W, b2d, Wmu, bmu2d, Wlv, blv2d):
    blk = 512
    nblk = N_PAD // blk
    lat = Wmu.shape[1]

    def body(t0r, t1r, t2r, t3r, w_r, b_r, wmu_r, bmu_r, wlv_r, blv_r,
             mu_ref, lv_ref, acc_ref):
        i = pl.program_id(0)

        @pl.when(i == 0)
        def _():
            acc_ref[...] = jnp.zeros_like(acc_ref)

        acc = jnp.dot(t0r[...], w_r[0], preferred_element_type=jnp.float32)
        acc = acc + jnp.dot(t1r[...], w_r[1], preferred_element_type=jnp.float32)
        acc = acc + jnp.dot(t2r[...], w_r[2], preferred_element_type=jnp.float32)
        acc = acc + jnp.dot(t3r[...], w_r[3], preferred_element_type=jnp.float32)
        h = jnp.maximum(acc + b_r[...], 0.0)
        row = i * blk + lax.broadcasted_iota(jnp.int32, (blk, 1), 0)
        h = jnp.where(row < N_NODES, h, 0.0)
        acc_ref[...] = acc_ref[...] + jnp.sum(h, axis=0, keepdims=True)

        @pl.when(i == nblk - 1)
        def _():
            ge = acc_ref[...] * (1.0 / N_NODES)
            mu_ref[...] = (
                jnp.dot(ge, wmu_r[...], preferred_element_type=jnp.float32)
                + bmu_r[...]
            )
            lv_ref[...] = (
                jnp.dot(ge, wlv_r[...], preferred_element_type=jnp.float32)
                + blv_r[...]
            )

    tspec = pl.BlockSpec((blk, 128), _im32(lambda i: (i, 0)))
    return pl.pallas_call(
        body,
        out_shape=(
            jax.ShapeDtypeStruct((1, lat), jnp.float32),
            jax.ShapeDtypeStruct((1, lat), jnp.float32),
        ),
        grid=(nblk,),
        in_specs=[
            tspec,
            tspec,
            tspec,
            tspec,
            pl.BlockSpec((4, 128, 128), _im32(lambda i: (0, 0, 0))),
            pl.BlockSpec((1, 128), _im32(lambda i: (0, 0))),
            pl.BlockSpec((128, lat), _im32(lambda i: (0, 0))),
            pl.BlockSpec((1, lat), _im32(lambda i: (0, 0))),
            pl.BlockSpec((128, lat), _im32(lambda i: (0, 0))),
            pl.BlockSpec((1, lat), _im32(lambda i: (0, 0))),
        ],
        out_specs=(
            pl.BlockSpec((1, lat), _im32(lambda i: (0, 0))),
            pl.BlockSpec((1, lat), _im32(lambda i: (0, 0))),
        ),
        scratch_shapes=[pltpu.VMEM((1, 128), jnp.float32)],
    )(t0, t1, t2, t3, W, b2d, Wmu, bmu2d, Wlv, blv2d)


def kernel(x, edge_index, lap_pe, edge_weight, W1, b1, W2, b2, Wmu, bmu, Wlv, blv):
    n = x.shape[0]
    e = edge_weight.shape[0]

    src = edge_index[0].astype(jnp.int32)
    dst = edge_index[1].astype(jnp.int32)
    loop = jnp.arange(n, dtype=jnp.int32)
    src = jnp.concatenate([src, loop])
    dst = jnp.concatenate([dst, loop])
    w = jnp.concatenate([edge_weight.astype(jnp.float32), jnp.ones((n,), jnp.float32)])

    ep = e + n
    nb = -(-ep // (NW * BATCH))
    pad = NW * nb * BATCH - ep
    src = jnp.concatenate([src, jnp.zeros((pad,), jnp.int32)])
    dst = jnp.concatenate([dst, jnp.full((pad,), N_NODES, jnp.int32)])
    w = jnp.concatenate([w, jnp.zeros((pad,), jnp.float32)])
    src3 = src.reshape(NW, nb, BATCH)
    dst3 = dst.reshape(NW, nb, BATCH)
    w3 = w.reshape(NW, nb, BATCH)

    degp = _deg_partials(src3, w3, nb)
    dis = _dis_tc(degp).reshape(N_PAD)
    what3 = _what_sc(src3, dst3, w3, dis, nb)

    xc = jnp.concatenate([x.astype(jnp.float32), lap_pe.astype(jnp.float32)], axis=1)
    din = xc.shape[1]
    t0 = jnp.pad(xc, ((0, N_PAD - n), (0, 0)))

    t1 = _combine0_tc(_matvec_sc(t0, src3, dst3, what3, nb, din), din)
    t2 = _combine_tc(_matvec_sc(t1, src3, dst3, what3, nb, din), t0, din)
    t3 = _combine_tc(_matvec_sc(t2, src3, dst3, what3, nb, din), t1, din)
    h = _layer1_tc(t0, t1, t2, t3, W1, b1.reshape(1, -1), din)

    u1 = _combine0_tc(_matvec_sc(h, src3, dst3, what3, nb, 128), 128)
    u2 = _combine_tc(_matvec_sc(u1, src3, dst3, what3, nb, 128), h, 128)
    u3 = _combine_tc(_matvec_sc(u2, src3, dst3, what3, nb, 128), u1, 128)

    mu, logvar = _layer2_tc(
        h, u1, u2, u3, W2, b2.reshape(1, -1),
        Wmu, bmu.reshape(1, -1), Wlv, blv.reshape(1, -1),
    )
    return (mu, logvar)


# speedup vs baseline: 11.5757x; 1.0957x over previous
import functools

import jax
import jax.numpy as jnp
import numpy as np
from jax import lax
from jax.experimental import pallas as pl
from jax.experimental.pallas import tpu as pltpu
from jax.experimental.pallas import tpu_sc as plsc

N_NODES = 10000
N_PAD = 10240
NUM_CORES = 2
NUM_SUBCORES = 16
NW = NUM_CORES * NUM_SUBCORES
BATCH = 64
ROWS_PER_SUB = N_PAD // NUM_SUBCORES


def _z():
    return jnp.int32(0)


def _im32(f):
    import functools as _ft

    @_ft.wraps(f)
    def g(*args):
        return tuple(jnp.asarray(v, jnp.int32) for v in f(*args))

    return g


def _mesh():
    return plsc.VectorSubcoreMesh(core_axis_name="c", subcore_axis_name="s")


def _wid():
    return lax.axis_index("s") * jnp.int32(NUM_CORES) + lax.axis_index("c")


def _deg_partials(src3, w3, nb):
    @functools.partial(
        pl.kernel,
        mesh=_mesh(),
        compiler_params=pltpu.CompilerParams(needs_layout_passes=False),
        out_type=jax.ShapeDtypeStruct((NUM_CORES, N_PAD), jnp.float32),
        scratch_types=[
            pltpu.VMEM((nb, BATCH), jnp.int32),
            pltpu.VMEM((nb, BATCH), jnp.float32),
            pltpu.VMEM((ROWS_PER_SUB,), jnp.float32),
            pltpu.VMEM_SHARED((N_PAD,), jnp.float32),
        ],
    )
    def k(src_h, w_h, out_h, src_v, w_v, z_v, deg_sp):
        sid = lax.axis_index("s")
        cid = lax.axis_index("c")
        wid = _wid()
        pltpu.sync_copy(src_h.at[wid], src_v)
        pltpu.sync_copy(w_h.at[wid], w_v)

        zero16 = jnp.zeros((16,), jnp.float32)

        def zfill(i, carry):
            z_v[pl.ds(i * jnp.int32(16), 16)] = zero16
            return carry

        lax.fori_loop(jnp.int32(0), jnp.int32(ROWS_PER_SUB // 16), zfill, jnp.int32(0))
        pltpu.sync_copy(z_v, deg_sp.at[pl.ds(sid * jnp.int32(ROWS_PER_SUB), ROWS_PER_SUB)])
        plsc.subcore_barrier()

        def jbody(j, carry):
            pltpu.sync_copy(w_v.at[j], deg_sp.at[src_v.at[j]], add=True)
            return carry

        lax.fori_loop(jnp.int32(0), jnp.int32(nb), jbody, jnp.int32(0))
        plsc.subcore_barrier()
        pltpu.sync_copy(
            deg_sp.at[pl.ds(sid * jnp.int32(ROWS_PER_SUB), ROWS_PER_SUB)],
            out_h.at[cid, pl.ds(sid * jnp.int32(ROWS_PER_SUB), ROWS_PER_SUB)],
        )

    return k(src3, w3)


def _dis_tc(degp):
    def body(d_ref, o_ref):
        deg = d_ref[0, :] + d_ref[1, :]
        o_ref[0, :] = jnp.where(deg > 0, lax.rsqrt(deg), 0.0)

    return pl.pallas_call(
        body, out_shape=jax.ShapeDtypeStruct((1, N_PAD), jnp.float32)
    )(degp)


def _what_sc(src3, dst3, w3, dis, nb):
    @functools.partial(
        pl.kernel,
        mesh=_mesh(),
        compiler_params=pltpu.CompilerParams(needs_layout_passes=False),
        out_type=jax.ShapeDtypeStruct((NW, nb, BATCH), jnp.float32),
        scratch_types=[
            pltpu.VMEM((N_PAD,), jnp.float32),
            pltpu.VMEM((nb, BATCH), jnp.int32),
            pltpu.VMEM((nb, BATCH), jnp.int32),
            pltpu.VMEM((nb, BATCH), jnp.float32),
            pltpu.VMEM((nb, BATCH), jnp.float32),
        ],
    )
    def k(src_h, dst_h, w_h, dis_h, out_h, dis_v, src_v, dst_v, w_v, o_v):
        wid = _wid()
        pltpu.sync_copy(dis_h, dis_v)
        pltpu.sync_copy(src_h.at[wid], src_v)
        pltpu.sync_copy(dst_h.at[wid], dst_v)
        pltpu.sync_copy(w_h.at[wid], w_v)

        def jbody(j, carry):
            for q in range(BATCH // 16):
                sl = pl.ds(q * 16, 16)
                s16 = src_v[j, sl]
                d16 = dst_v[j, sl]
                wv = w_v[j, sl]
                sv = plsc.load_gather(dis_v, [s16])
                dv = plsc.load_gather(dis_v, [d16])
                o_v[j, sl] = -(sv * wv * dv)
            return carry

        lax.fori_loop(jnp.int32(0), jnp.int32(nb), jbody, jnp.int32(0))
        pltpu.sync_copy(o_v, out_h.at[wid])

    return k(src3, dst3, w3, dis)


def _matvec_sc(t, src3, dst3, what3, nb, d):
    ch_n = 54
    assert nb % ch_n == 0 and ch_n % 3 == 0
    nch = nb // ch_n
    ntr = ch_n // 3
    ng = BATCH // 16
    i32 = jnp.int32

    @functools.partial(
        pl.kernel,
        mesh=_mesh(),
        compiler_params=pltpu.CompilerParams(
            needs_layout_passes=False, use_tc_tiling_on_sc=False
        ),
        out_type=jax.ShapeDtypeStruct((NUM_CORES, N_PAD, d), jnp.float32),
        scratch_types=[
            pltpu.VMEM((ch_n, BATCH), jnp.int32),
            pltpu.VMEM((ch_n, BATCH), jnp.int32),
            pltpu.VMEM((ch_n, BATCH), jnp.float32),
            pltpu.VMEM((BATCH, d), jnp.float32),
            pltpu.VMEM((BATCH, d), jnp.float32),
            pltpu.VMEM((BATCH, d), jnp.float32),
            pltpu.VMEM_SHARED((N_PAD, d), jnp.float32),
            pltpu.SemaphoreType.DMA,
            pltpu.SemaphoreType.DMA,
            pltpu.SemaphoreType.DMA,
            pltpu.SemaphoreType.DMA,
            pltpu.SemaphoreType.DMA,
            pltpu.SemaphoreType.DMA,
        ],
    )
    def k(t_h, src_h, dst_h, w_h, out_h,
          src_v, dst_v, w_v, r0, r1, r2, acc_sp, g0, g1, g2, s0, s1, s2):
        sid = lax.axis_index("s")
        cid = lax.axis_index("c")
        wid = _wid()
        rows = (r0, r1, r2)
        gsem = (g0, g1, g2)
        ssem = (s0, s1, s2)

        zero16 = jnp.zeros((16,), jnp.float32)

        def zfill(r, carry):
            for q in range(d // 16):
                r0[r, pl.ds(q * 16, 16)] = zero16
            return carry

        lax.fori_loop(i32(0), i32(BATCH), zfill, i32(0))

        def zcopy(m, carry):
            pltpu.sync_copy(
                r0,
                acc_sp.at[pl.ds(sid * i32(ROWS_PER_SUB) + m * i32(BATCH), BATCH)],
            )
            return carry

        assert ROWS_PER_SUB % BATCH == 0
        lax.fori_loop(i32(0), i32(ROWS_PER_SUB // BATCH), zcopy, i32(0))
        plsc.subcore_barrier()

        def load_chunk(c):
            csl = pl.ds(c * i32(ch_n), ch_n)
            pltpu.sync_copy(src_h.at[wid, csl], src_v)
            pltpu.sync_copy(dst_h.at[wid, csl], dst_v)
            pltpu.sync_copy(w_h.at[wid, csl], w_v)

        def g_start(jj, b):
            pltpu.async_copy(t_h.at[src_v.at[jj]], rows[b], gsem[b])

        def g_wait(jj, b):
            pltpu.make_async_copy(t_h.at[src_v.at[jj]], rows[b], gsem[b]).wait()

        def s_start(jj, b):
            pltpu.async_copy(rows[b], acc_sp.at[dst_v.at[jj]], ssem[b], add=True)

        def s_wait(jj, b):
            pltpu.make_async_copy(rows[b], acc_sp.at[dst_v.at[jj]], ssem[b]).wait()

        zi16 = lax.broadcasted_iota(jnp.int32, (16,), 0) * i32(0)

        def scale(b, jj):
            idx_j = zi16 + jj

            def gbody(g, carry):
                g16 = g * i32(16)
                for e in range(16):
                    r = g16 + i32(e)
                    wb = plsc.load_gather(w_v, [idx_j, zi16 + r])
                    for q in range(d // 16):
                        sl = pl.ds(q * 16, 16)
                        rows[b][r, sl] = rows[b][r, sl] * wb
                return carry

            lax.fori_loop(i32(0), i32(ng), gbody, i32(0))

        load_chunk(i32(0))
        g_start(i32(0), 0)
        g_start(i32(1), 1)

        def tbody(tr, carry):
            trm = lax.rem(tr, i32(ntr))

            @pl.when(jnp.logical_and(trm == 0, tr > 0))
            def _():
                s_wait(i32(ch_n - 3), 0)
                s_wait(i32(ch_n - 2), 1)
                s_wait(i32(ch_n - 1), 2)
                load_chunk(lax.div(tr, i32(ntr)))
                g_start(i32(0), 0)
                g_start(i32(1), 1)

            jbase = trm * i32(3)
            for o in range(3):
                jj = jbase + i32(o)
                g_wait(jj, o)
                scale(o, jj)
                s_start(jj, o)
                z = (o + 2) % 3
                if o == 0:
                    @pl.when(jbase > 0)
                    def _(jj=jj, z=z):
                        s_wait(jj - i32(1), z)
                        g_start(jj + i32(2), z)

                    @pl.when(jbase == 0)
                    def _(jj=jj, z=z):
                        g_start(jj + i32(2), z)
                else:

                    @pl.when(jj + i32(2) < i32(ch_n))
                    def _(jj=jj, z=z):
                        s_wait(jj - i32(1), z)
                        g_start(jj + i32(2), z)

            return carry

        lax.fori_loop(i32(0), i32(nch * ntr), tbody, i32(0))
        s_wait(i32(ch_n - 3), 0)
        s_wait(i32(ch_n - 2), 1)
        s_wait(i32(ch_n - 1), 2)
        plsc.subcore_barrier()
        pltpu.sync_copy(
            acc_sp.at[pl.ds(sid * i32(ROWS_PER_SUB), ROWS_PER_SUB)],
            out_h.at[cid, pl.ds(sid * i32(ROWS_PER_SUB), ROWS_PER_SUB)],
        )

    return k(t, src3, dst3, what3)


def _combine0_tc(parts, d):
    blk = 1024

    def body(p_ref, o_ref):
        o_ref[...] = p_ref[0] + p_ref[1]

    return pl.pallas_call(
        body,
        out_shape=jax.ShapeDtypeStruct((N_PAD, d), jnp.float32),
        grid=(N_PAD // blk,),
        in_specs=[pl.BlockSpec((2, blk, d), _im32(lambda i: (0, i, 0)))],
        out_specs=pl.BlockSpec((blk, d), _im32(lambda i: (i, 0))),
    )(parts)


def _combine_tc(parts, tprev, d):
    blk = 1024

    def body(p_ref, tp_ref, o_ref):
        o_ref[...] = 2.0 * (p_ref[0] + p_ref[1]) - tp_ref[...]

    return pl.pallas_call(
        body,
        out_shape=jax.ShapeDtypeStruct((N_PAD, d), jnp.float32),
        grid=(N_PAD // blk,),
        in_specs=[
            pl.BlockSpec((2, blk, d), _im32(lambda i: (0, i, 0))),
            pl.BlockSpec((blk, d), _im32(lambda i: (i, 0))),
        ],
        out_specs=pl.BlockSpec((blk, d), _im32(lambda i: (i, 0))),
    )(parts, tprev)


def _layer1_tc(t0, t1, t2, t3, W, b2d, din):
    blk = 512

    def body(t0r, t1r, t2r, t3r, w_r, b_r, o_ref):
        acc = jnp.dot(t0r[...], w_r[0], preferred_element_type=jnp.float32)
        acc = acc + jnp.dot(t1r[...], w_r[1], preferred_element_type=jnp.float32)
        acc = acc + jnp.dot(t2r[...], w_r[2], preferred_element_type=jnp.float32)
        acc = acc + jnp.dot(t3r[...], w_r[3], preferred_element_type=jnp.float32)
        o_ref[...] = jnp.maximum(acc + b_r[...], 0.0)

    tspec = pl.BlockSpec((blk, din), _im32(lambda i: (i, 0)))
    return pl.pallas_call(
        body,
        out_shape=jax.ShapeDtypeStruct((N_PAD, 128), jnp.float32),
        grid=(N_PAD // blk,),
        in_specs=[
            tspec,
            tspec,
            tspec,
            tspec,
            pl.BlockSpec((4, din, 128), _im32(lambda i: (0, 0, 0))),
            pl.BlockSpec((1, 128), _im32(lambda i: (0, 0))),
        ],
        out_specs=pl.BlockSpec((blk, 128), _im32(lambda i: (i, 0))),
    )(t0, t1, t2, t3, W, b2d)


def _layer2_tc(t0, t1, t2, t3, W, b2d, Wmu, bmu2d, Wlv, blv2d):
    blk = 512
    nblk = N_PAD // blk
    lat = Wmu.shape[1]

    def body(t0r, t1r, t2r, t3r, w_r, b_r, wmu_r, bmu_r, wlv_r, blv_r,
             mu_ref, lv_ref, acc_ref):
        i = pl.program_id(0)

        @pl.when(i == 0)
        def _():
            acc_ref[...] = jnp.zeros_like(acc_ref)

        acc = jnp.dot(t0r[...], w_r[0], preferred_element_type=jnp.float32)
        acc = acc + jnp.dot(t1r[...], w_r[1], preferred_element_type=jnp.float32)
        acc = acc + jnp.dot(t2r[...], w_r[2], preferred_element_type=jnp.float32)
        acc = acc + jnp.dot(t3r[...], w_r[3], preferred_element_type=jnp.float32)
        h = jnp.maximum(acc + b_r[...], 0.0)
        row = i * blk + lax.broadcasted_iota(jnp.int32, (blk, 1), 0)
        h = jnp.where(row < N_NODES, h, 0.0)
        acc_ref[...] = acc_ref[...] + jnp.sum(h, axis=0, keepdims=True)

        @pl.when(i == nblk - 1)
        def _():
            ge = acc_ref[...] * (1.0 / N_NODES)
            mu_ref[...] = (
                jnp.dot(ge, wmu_r[...], preferred_element_type=jnp.float32)
                + bmu_r[...]
            )
            lv_ref[...] = (
                jnp.dot(ge, wlv_r[...], preferred_element_type=jnp.float32)
                + blv_r[...]
            )

    tspec = pl.BlockSpec((blk, 128), _im32(lambda i: (i, 0)))
    return pl.pallas_call(
        body,
        out_shape=(
            jax.ShapeDtypeStruct((1, lat), jnp.float32),
            jax.ShapeDtypeStruct((1, lat), jnp.float32),
        ),
        grid=(nblk,),
        in_specs=[
            tspec,
            tspec,
            tspec,
            tspec,
            pl.BlockSpec((4, 128, 128), _im32(lambda i: (0, 0, 0))),
            pl.BlockSpec((1, 128), _im32(lambda i: (0, 0))),
            pl.BlockSpec((128, lat), _im32(lambda i: (0, 0))),
            pl.BlockSpec((1, lat), _im32(lambda i: (0, 0))),
            pl.BlockSpec((128, lat), _im32(lambda i: (0, 0))),
            pl.BlockSpec((1, lat), _im32(lambda i: (0, 0))),
        ],
        out_specs=(
            pl.BlockSpec((1, lat), _im32(lambda i: (0, 0))),
            pl.BlockSpec((1, lat), _im32(lambda i: (0, 0))),
        ),
        scratch_shapes=[pltpu.VMEM((1, 128), jnp.float32)],
    )(t0, t1, t2, t3, W, b2d, Wmu, bmu2d, Wlv, blv2d)


def kernel(x, edge_index, lap_pe, edge_weight, W1, b1, W2, b2, Wmu, bmu, Wlv, blv):
    n = x.shape[0]
    e = edge_weight.shape[0]

    src = edge_index[0].astype(jnp.int32)
    dst = edge_index[1].astype(jnp.int32)
    loop = jnp.arange(n, dtype=jnp.int32)
    src = jnp.concatenate([src, loop])
    dst = jnp.concatenate([dst, loop])
    w = jnp.concatenate([edge_weight.astype(jnp.float32), jnp.ones((n,), jnp.float32)])

    ep = e + n
    nb = -(-ep // (NW * BATCH))
    pad = NW * nb * BATCH - ep
    src = jnp.concatenate([src, jnp.zeros((pad,), jnp.int32)])
    dst = jnp.concatenate([dst, jnp.full((pad,), N_NODES, jnp.int32)])
    w = jnp.concatenate([w, jnp.zeros((pad,), jnp.float32)])
    src3 = src.reshape(NW, nb, BATCH)
    dst3 = dst.reshape(NW, nb, BATCH)
    w3 = w.reshape(NW, nb, BATCH)

    degp = _deg_partials(src3, w3, nb)
    dis = _dis_tc(degp).reshape(N_PAD)
    what3 = _what_sc(src3, dst3, w3, dis, nb)

    xc = jnp.concatenate([x.astype(jnp.float32), lap_pe.astype(jnp.float32)], axis=1)
    din = xc.shape[1]
    t0 = jnp.pad(xc, ((0, N_PAD - n), (0, 0)))

    t1 = _combine0_tc(_matvec_sc(t0, src3, dst3, what3, nb, din), din)
    t2 = _combine_tc(_matvec_sc(t1, src3, dst3, what3, nb, din), t0, din)
    t3 = _combine_tc(_matvec_sc(t2, src3, dst3, what3, nb, din), t1, din)
    h = _layer1_tc(t0, t1, t2, t3, W1, b1.reshape(1, -1), din)

    u1 = _combine0_tc(_matvec_sc(h, src3, dst3, what3, nb, 128), 128)
    u2 = _combine_tc(_matvec_sc(u1, src3, dst3, what3, nb, 128), h, 128)
    u3 = _combine_tc(_matvec_sc(u2, src3, dst3, what3, nb, 128), u1, 128)

    mu, logvar = _layer2_tc(
        h, u1, u2, u3, W2, b2.reshape(1, -1),
        Wmu, bmu.reshape(1, -1), Wlv, blv.reshape(1, -1),
    )
    return (mu, logvar)


# speedup vs baseline: 11.5855x; 1.0009x over previous
import functools

import jax
import jax.numpy as jnp
import numpy as np
from jax import lax
from jax.experimental import pallas as pl
from jax.experimental.pallas import tpu as pltpu
from jax.experimental.pallas import tpu_sc as plsc

N_NODES = 10000
N_PAD = 10240
NUM_CORES = 2
NUM_SUBCORES = 16
NW = NUM_CORES * NUM_SUBCORES
BATCH = 64
ROWS_PER_SUB = N_PAD // NUM_SUBCORES


def _z():
    return jnp.int32(0)


def _im32(f):
    import functools as _ft

    @_ft.wraps(f)
    def g(*args):
        return tuple(jnp.asarray(v, jnp.int32) for v in f(*args))

    return g


def _mesh():
    return plsc.VectorSubcoreMesh(core_axis_name="c", subcore_axis_name="s")


def _wid():
    return lax.axis_index("s") * jnp.int32(NUM_CORES) + lax.axis_index("c")


def _deg_partials(src3, w3, nb):
    @functools.partial(
        pl.kernel,
        mesh=_mesh(),
        compiler_params=pltpu.CompilerParams(needs_layout_passes=False),
        out_type=jax.ShapeDtypeStruct((NUM_CORES, N_PAD), jnp.float32),
        scratch_types=[
            pltpu.VMEM((nb, BATCH), jnp.int32),
            pltpu.VMEM((nb, BATCH), jnp.float32),
            pltpu.VMEM((ROWS_PER_SUB,), jnp.float32),
            pltpu.VMEM_SHARED((N_PAD,), jnp.float32),
        ],
    )
    def k(src_h, w_h, out_h, src_v, w_v, z_v, deg_sp):
        sid = lax.axis_index("s")
        cid = lax.axis_index("c")
        wid = _wid()
        pltpu.sync_copy(src_h.at[wid], src_v)
        pltpu.sync_copy(w_h.at[wid], w_v)

        zero16 = jnp.zeros((16,), jnp.float32)

        def zfill(i, carry):
            z_v[pl.ds(i * jnp.int32(16), 16)] = zero16
            return carry

        lax.fori_loop(jnp.int32(0), jnp.int32(ROWS_PER_SUB // 16), zfill, jnp.int32(0))
        pltpu.sync_copy(z_v, deg_sp.at[pl.ds(sid * jnp.int32(ROWS_PER_SUB), ROWS_PER_SUB)])
        plsc.subcore_barrier()

        def jbody(j, carry):
            pltpu.sync_copy(w_v.at[j], deg_sp.at[src_v.at[j]], add=True)
            return carry

        lax.fori_loop(jnp.int32(0), jnp.int32(nb), jbody, jnp.int32(0))
        plsc.subcore_barrier()
        pltpu.sync_copy(
            deg_sp.at[pl.ds(sid * jnp.int32(ROWS_PER_SUB), ROWS_PER_SUB)],
            out_h.at[cid, pl.ds(sid * jnp.int32(ROWS_PER_SUB), ROWS_PER_SUB)],
        )

    return k(src3, w3)


def _what_sc(src3, dst3, w3, degp, nb):
    @functools.partial(
        pl.kernel,
        mesh=_mesh(),
        compiler_params=pltpu.CompilerParams(needs_layout_passes=False),
        out_type=jax.ShapeDtypeStruct((NW, nb, BATCH), jnp.float32),
        scratch_types=[
            pltpu.VMEM((N_PAD,), jnp.float32),
            pltpu.VMEM((NUM_CORES, N_PAD), jnp.float32),
            pltpu.VMEM((nb, BATCH), jnp.int32),
            pltpu.VMEM((nb, BATCH), jnp.int32),
            pltpu.VMEM((nb, BATCH), jnp.float32),
            pltpu.VMEM((nb, BATCH), jnp.float32),
        ],
    )
    def k(src_h, dst_h, w_h, degp_h, out_h, dis_v, degp_v, src_v, dst_v, w_v, o_v):
        wid = _wid()
        pltpu.sync_copy(degp_h, degp_v)
        pltpu.sync_copy(src_h.at[wid], src_v)
        pltpu.sync_copy(dst_h.at[wid], dst_v)
        pltpu.sync_copy(w_h.at[wid], w_v)

        i32 = jnp.int32
        zero16 = jnp.zeros((16,), jnp.float32)

        def dbody(i, carry):
            sl = pl.ds(i * i32(16), 16)
            x = degp_v[0, sl] + degp_v[1, sl]
            bi = i32(0x5F3759DF) - lax.shift_right_arithmetic(
                plsc.bitcast(x, jnp.int32), i32(1)
            )
            y = plsc.bitcast(bi, jnp.float32)
            hx = x * 0.5
            y = y * (1.5 - hx * y * y)
            y = y * (1.5 - hx * y * y)
            y = y * (1.5 - hx * y * y)
            dis_v[sl] = jnp.where(x > 0, y, zero16)
            return carry

        lax.fori_loop(i32(0), i32(N_PAD // 16), dbody, i32(0))

        def jbody(j, carry):
            for q in range(BATCH // 16):
                sl = pl.ds(q * 16, 16)
                s16 = src_v[j, sl]
                d16 = dst_v[j, sl]
                wv = w_v[j, sl]
                sv = plsc.load_gather(dis_v, [s16])
                dv = plsc.load_gather(dis_v, [d16])
                o_v[j, sl] = -(sv * wv * dv)
            return carry

        lax.fori_loop(i32(0), i32(nb), jbody, i32(0))
        pltpu.sync_copy(o_v, out_h.at[wid])

    return k(src3, dst3, w3, degp)


def _matvec_sc(t, src3, dst3, what3, nb, d):
    ch_n = 54
    assert nb % ch_n == 0 and ch_n % 3 == 0
    nch = nb // ch_n
    ntr = ch_n // 3
    ng = BATCH // 16
    i32 = jnp.int32

    @functools.partial(
        pl.kernel,
        mesh=_mesh(),
        compiler_params=pltpu.CompilerParams(
            needs_layout_passes=False, use_tc_tiling_on_sc=False
        ),
        out_type=jax.ShapeDtypeStruct((NUM_CORES, N_PAD, d), jnp.float32),
        scratch_types=[
            pltpu.VMEM((ch_n, BATCH), jnp.int32),
            pltpu.VMEM((ch_n, BATCH), jnp.int32),
            pltpu.VMEM((ch_n, BATCH), jnp.float32),
            pltpu.VMEM((BATCH, d), jnp.float32),
            pltpu.VMEM((BATCH, d), jnp.float32),
            pltpu.VMEM((BATCH, d), jnp.float32),
            pltpu.VMEM_SHARED((N_PAD, d), jnp.float32),
            pltpu.SemaphoreType.DMA,
            pltpu.SemaphoreType.DMA,
            pltpu.SemaphoreType.DMA,
            pltpu.SemaphoreType.DMA,
            pltpu.SemaphoreType.DMA,
            pltpu.SemaphoreType.DMA,
        ],
    )
    def k(t_h, src_h, dst_h, w_h, out_h,
          src_v, dst_v, w_v, r0, r1, r2, acc_sp, g0, g1, g2, s0, s1, s2):
        sid = lax.axis_index("s")
        cid = lax.axis_index("c")
        wid = _wid()
        rows = (r0, r1, r2)
        gsem = (g0, g1, g2)
        ssem = (s0, s1, s2)

        zero16 = jnp.zeros((16,), jnp.float32)

        def zfill(r, carry):
            for q in range(d // 16):
                r0[r, pl.ds(q * 16, 16)] = zero16
            return carry

        lax.fori_loop(i32(0), i32(BATCH), zfill, i32(0))

        def zcopy(m, carry):
            pltpu.sync_copy(
                r0,
                acc_sp.at[pl.ds(sid * i32(ROWS_PER_SUB) + m * i32(BATCH), BATCH)],
            )
            return carry

        assert ROWS_PER_SUB % BATCH == 0
        lax.fori_loop(i32(0), i32(ROWS_PER_SUB // BATCH), zcopy, i32(0))
        plsc.subcore_barrier()

        def load_chunk(c):
            csl = pl.ds(c * i32(ch_n), ch_n)
            pltpu.sync_copy(src_h.at[wid, csl], src_v)
            pltpu.sync_copy(dst_h.at[wid, csl], dst_v)
            pltpu.sync_copy(w_h.at[wid, csl], w_v)

        def g_start(jj, b):
            pltpu.async_copy(t_h.at[src_v.at[jj]], rows[b], gsem[b])

        def g_wait(jj, b):
            pltpu.make_async_copy(t_h.at[src_v.at[jj]], rows[b], gsem[b]).wait()

        def s_start(jj, b):
            pltpu.async_copy(rows[b], acc_sp.at[dst_v.at[jj]], ssem[b], add=True)

        def s_wait(jj, b):
            pltpu.make_async_copy(rows[b], acc_sp.at[dst_v.at[jj]], ssem[b]).wait()

        zi16 = lax.broadcasted_iota(jnp.int32, (16,), 0) * i32(0)

        def scale(b, jj):
            idx_j = zi16 + jj

            def gbody(g, carry):
                g16 = g * i32(16)
                for e in range(16):
                    r = g16 + i32(e)
                    wb = plsc.load_gather(w_v, [idx_j, zi16 + r])
                    for q in range(d // 16):
                        sl = pl.ds(q * 16, 16)
                        rows[b][r, sl] = rows[b][r, sl] * wb
                return carry

            lax.fori_loop(i32(0), i32(ng), gbody, i32(0))

        load_chunk(i32(0))
        g_start(i32(0), 0)
        g_start(i32(1), 1)

        def tbody(tr, carry):
            trm = lax.rem(tr, i32(ntr))

            @pl.when(jnp.logical_and(trm == 0, tr > 0))
            def _():
                s_wait(i32(ch_n - 3), 0)
                s_wait(i32(ch_n - 2), 1)
                s_wait(i32(ch_n - 1), 2)
                load_chunk(lax.div(tr, i32(ntr)))
                g_start(i32(0), 0)
                g_start(i32(1), 1)

            jbase = trm * i32(3)
            for o in range(3):
                jj = jbase + i32(o)
                g_wait(jj, o)
                scale(o, jj)
                s_start(jj, o)
                z = (o + 2) % 3
                if o == 0:
                    @pl.when(jbase > 0)
                    def _(jj=jj, z=z):
                        s_wait(jj - i32(1), z)
                        g_start(jj + i32(2), z)

                    @pl.when(jbase == 0)
                    def _(jj=jj, z=z):
                        g_start(jj + i32(2), z)
                else:

                    @pl.when(jj + i32(2) < i32(ch_n))
                    def _(jj=jj, z=z):
                        s_wait(jj - i32(1), z)
                        g_start(jj + i32(2), z)

            return carry

        lax.fori_loop(i32(0), i32(nch * ntr), tbody, i32(0))
        s_wait(i32(ch_n - 3), 0)
        s_wait(i32(ch_n - 2), 1)
        s_wait(i32(ch_n - 1), 2)
        plsc.subcore_barrier()
        pltpu.sync_copy(
            acc_sp.at[pl.ds(sid * i32(ROWS_PER_SUB), ROWS_PER_SUB)],
            out_h.at[cid, pl.ds(sid * i32(ROWS_PER_SUB), ROWS_PER_SUB)],
        )

    return k(t, src3, dst3, what3)


def _combine0_tc(parts, d):
    blk = 1024

    def body(p_ref, o_ref):
        o_ref[...] = p_ref[0] + p_ref[1]

    return pl.pallas_call(
        body,
        out_shape=jax.ShapeDtypeStruct((N_PAD, d), jnp.float32),
        grid=(N_PAD // blk,),
        in_specs=[pl.BlockSpec((2, blk, d), _im32(lambda i: (0, i, 0)))],
        out_specs=pl.BlockSpec((blk, d), _im32(lambda i: (i, 0))),
    )(parts)


def _combine_tc(parts, tprev, d):
    blk = 1024

    def body(p_ref, tp_ref, o_ref):
        o_ref[...] = 2.0 * (p_ref[0] + p_ref[1]) - tp_ref[...]

    return pl.pallas_call(
        body,
        out_shape=jax.ShapeDtypeStruct((N_PAD, d), jnp.float32),
        grid=(N_PAD // blk,),
        in_specs=[
            pl.BlockSpec((2, blk, d), _im32(lambda i: (0, i, 0))),
            pl.BlockSpec((blk, d), _im32(lambda i: (i, 0))),
        ],
        out_specs=pl.BlockSpec((blk, d), _im32(lambda i: (i, 0))),
    )(parts, tprev)


def _layer1_tc(t0, t1, t2, t3parts, W, b2d, din):
    blk = 512

    def body(t0r, t1r, t2r, p3r, w_r, b_r, o_ref):
        t3 = 2.0 * (p3r[0] + p3r[1]) - t1r[...]
        acc = jnp.dot(t0r[...], w_r[0], preferred_element_type=jnp.float32)
        acc = acc + jnp.dot(t1r[...], w_r[1], preferred_element_type=jnp.float32)
        acc = acc + jnp.dot(t2r[...], w_r[2], preferred_element_type=jnp.float32)
        acc = acc + jnp.dot(t3, w_r[3], preferred_element_type=jnp.float32)
        o_ref[...] = jnp.maximum(acc + b_r[...], 0.0)

    tspec = pl.BlockSpec((blk, din), _im32(lambda i: (i, 0)))
    return pl.pallas_call(
        body,
        out_shape=jax.ShapeDtypeStruct((N_PAD, 128), jnp.float32),
        grid=(N_PAD // blk,),
        in_specs=[
            tspec,
            tspec,
            tspec,
            pl.BlockSpec((2, blk, din), _im32(lambda i: (0, i, 0))),
            pl.BlockSpec((4, din, 128), _im32(lambda i: (0, 0, 0))),
            pl.BlockSpec((1, 128), _im32(lambda i: (0, 0))),
        ],
        out_specs=pl.BlockSpec((blk, 128), _im32(lambda i: (i, 0))),
    )(t0, t1, t2, t3parts, W, b2d)


def _layer2_tc(t0, t1, t2, t3parts, W, b2d, Wmu, bmu2d, Wlv, blv2d):
    blk = 512
    nblk = N_PAD // blk
    lat = Wmu.shape[1]

    def body(t0r, t1r, t2r, p3r, w_r, b_r, wmu_r, bmu_r, wlv_r, blv_r,
             mu_ref, lv_ref, acc_ref):
        i = pl.program_id(0)

        @pl.when(i == 0)
        def _():
            acc_ref[...] = jnp.zeros_like(acc_ref)

        t3 = 2.0 * (p3r[0] + p3r[1]) - t1r[...]
        acc = jnp.dot(t0r[...], w_r[0], preferred_element_type=jnp.float32)
        acc = acc + jnp.dot(t1r[...], w_r[1], preferred_element_type=jnp.float32)
        acc = acc + jnp.dot(t2r[...], w_r[2], preferred_element_type=jnp.float32)
        acc = acc + jnp.dot(t3, w_r[3], preferred_element_type=jnp.float32)
        h = jnp.maximum(acc + b_r[...], 0.0)
        row = i * blk + lax.broadcasted_iota(jnp.int32, (blk, 1), 0)
        h = jnp.where(row < N_NODES, h, 0.0)
        acc_ref[...] = acc_ref[...] + jnp.sum(h, axis=0, keepdims=True)

        @pl.when(i == nblk - 1)
        def _():
            ge = acc_ref[...] * (1.0 / N_NODES)
            mu_ref[...] = (
                jnp.dot(ge, wmu_r[...], preferred_element_type=jnp.float32)
                + bmu_r[...]
            )
            lv_ref[...] = (
                jnp.dot(ge, wlv_r[...], preferred_element_type=jnp.float32)
                + blv_r[...]
            )

    tspec = pl.BlockSpec((blk, 128), _im32(lambda i: (i, 0)))
    return pl.pallas_call(
        body,
        out_shape=(
            jax.ShapeDtypeStruct((1, lat), jnp.float32),
            jax.ShapeDtypeStruct((1, lat), jnp.float32),
        ),
        grid=(nblk,),
        in_specs=[
            tspec,
            tspec,
            tspec,
            pl.BlockSpec((2, blk, 128), _im32(lambda i: (0, i, 0))),
            pl.BlockSpec((4, 128, 128), _im32(lambda i: (0, 0, 0))),
            pl.BlockSpec((1, 128), _im32(lambda i: (0, 0))),
            pl.BlockSpec((128, lat), _im32(lambda i: (0, 0))),
            pl.BlockSpec((1, lat), _im32(lambda i: (0, 0))),
            pl.BlockSpec((128, lat), _im32(lambda i: (0, 0))),
            pl.BlockSpec((1, lat), _im32(lambda i: (0, 0))),
        ],
        out_specs=(
            pl.BlockSpec((1, lat), _im32(lambda i: (0, 0))),
            pl.BlockSpec((1, lat), _im32(lambda i: (0, 0))),
        ),
        scratch_shapes=[pltpu.VMEM((1, 128), jnp.float32)],
    )(t0, t1, t2, t3parts, W, b2d, Wmu, bmu2d, Wlv, blv2d)


def kernel(x, edge_index, lap_pe, edge_weight, W1, b1, W2, b2, Wmu, bmu, Wlv, blv):
    n = x.shape[0]
    e = edge_weight.shape[0]

    src = edge_index[0].astype(jnp.int32)
    dst = edge_index[1].astype(jnp.int32)
    loop = jnp.arange(n, dtype=jnp.int32)
    src = jnp.concatenate([src, loop])
    dst = jnp.concatenate([dst, loop])
    w = jnp.concatenate([edge_weight.astype(jnp.float32), jnp.ones((n,), jnp.float32)])

    ep = e + n
    nb = -(-ep // (NW * BATCH))
    pad = NW * nb * BATCH - ep
    src = jnp.concatenate([src, jnp.zeros((pad,), jnp.int32)])
    dst = jnp.concatenate([dst, jnp.full((pad,), N_NODES, jnp.int32)])
    w = jnp.concatenate([w, jnp.zeros((pad,), jnp.float32)])
    src3 = src.reshape(NW, nb, BATCH)
    dst3 = dst.reshape(NW, nb, BATCH)
    w3 = w.reshape(NW, nb, BATCH)

    degp = _deg_partials(src3, w3, nb)
    what3 = _what_sc(src3, dst3, w3, degp, nb)

    xc = jnp.concatenate([x.astype(jnp.float32), lap_pe.astype(jnp.float32)], axis=1)
    din = xc.shape[1]
    t0 = jnp.pad(xc, ((0, N_PAD - n), (0, 0)))

    t1 = _combine0_tc(_matvec_sc(t0, src3, dst3, what3, nb, din), din)
    t2 = _combine_tc(_matvec_sc(t1, src3, dst3, what3, nb, din), t0, din)
    p3 = _matvec_sc(t2, src3, dst3, what3, nb, din)
    h = _layer1_tc(t0, t1, t2, p3, W1, b1.reshape(1, -1), din)

    u1 = _combine0_tc(_matvec_sc(h, src3, dst3, what3, nb, 128), 128)
    u2 = _combine_tc(_matvec_sc(u1, src3, dst3, what3, nb, 128), h, 128)
    q3 = _matvec_sc(u2, src3, dst3, what3, nb, 128)

    mu, logvar = _layer2_tc(
        h, u1, u2, q3, W2, b2.reshape(1, -1),
        Wmu, bmu.reshape(1, -1), Wlv, blv.reshape(1, -1),
    )
    return (mu, logvar)
